# baseline (speedup 1.0000x reference)
import math

import jax
import jax.numpy as jnp
from jax import lax
from jax.experimental import pallas as pl
from jax.experimental.pallas import tpu as pltpu
from jax.experimental.pallas import tpu_sc as plsc

N = 10000
E = 320000
D = 128
H = 128
OUT = 128
AVG_LOG_DEG = float((math.log(1.0) + math.log(2.0)) / 2.0)

NC, NS = 2, 16
GCH = 64
N_OUT = 10240
N_TILE = N_OUT // NS
ACC_ROWS = N_OUT + 8
N_DEG = N_OUT // NC
DEG_ROWS = N_DEG + 8
DEG_TILE = N_DEG // NS

CPT = 320
GRP = 16
NGRP = CPT // GRP
E_PAD = CPT * NS * GCH
IDX_ROWS = E_PAD // GCH


def _pre_kernel(x_ref, w_ref, b_ref, o_ref):
    acc = jnp.dot(x_ref[...], w_ref[...],
                  preferred_element_type=jnp.float32,
                  precision=jax.lax.Precision.HIGHEST)
    o_ref[...] = acc + b_ref[...][None, :]


def _pre_project(x, W_pre, b_pre):
    blk = 1000
    grid = (N // blk,)
    return pl.pallas_call(
        _pre_kernel,
        grid=grid,
        in_specs=[
            pl.BlockSpec((blk, D), lambda i: (i, 0)),
            pl.BlockSpec((D, H), lambda i: (0, 0)),
            pl.BlockSpec((H,), lambda i: (0,)),
        ],
        out_specs=pl.BlockSpec((blk, H), lambda i: (i, 0)),
        out_shape=jax.ShapeDtypeStruct((N_OUT, H), jnp.float32),
    )(x, W_pre, b_pre)


def _square_rows(buf, nrows):
    def sq_row(i, _):
        for q in range(H // 16):
            v = buf[i, pl.ds(q * 16, 16)]
            buf[i, pl.ds(q * 16, 16)] = v * v
        return ()
    lax.fori_loop(0, nrows, sq_row, (), unroll=2)


def _sc_body(h_ref, src_ref, dst_ref, s_out, sq_out, deg_out,
             acc_main, acc_deg, src_buf, dst_buf, dstl_buf,
             rows_a, rows_b, rows_c, ones16,
             sem_ga, sem_gb, sem_gc, sem_pa, sem_pb, sem_pc, sem_d):
    c = lax.axis_index("c")
    t = lax.axis_index("s")

    def fill_ones(i, _):
        ones16[i, :] = jnp.full((16,), 1.0, jnp.float32)
        return ()
    lax.fori_loop(0, GCH, fill_ones, (), unroll=4)

    r0 = t * N_TILE
    d0 = t * DEG_TILE
    lo = c * N_DEG

    def run_core(do_square):
        def init_sub(k, _):
            rs = r0 + k * GCH
            pltpu.sync_copy(h_ref.at[pl.ds(rs, GCH)], rows_a)
            if do_square:
                _square_rows(rows_a, GCH)
            pltpu.sync_copy(rows_a, acc_main.at[pl.ds(rs, GCH)])
            return ()
        lax.fori_loop(0, N_TILE // GCH, init_sub, ())

        def init_deg(k, _):
            pltpu.sync_copy(ones16.at[pl.ds(0, 64)],
                            acc_deg.at[pl.ds(d0 + k * 64, 64)])
            return ()
        lax.fori_loop(0, DEG_TILE // 64, init_deg, ())

        plsc.subcore_barrier()

        def group_body(grp, _):
            base = t * CPT + grp * GRP
            pltpu.sync_copy(src_ref.at[pl.ds(base, GRP)], src_buf)
            pltpu.sync_copy(dst_ref.at[pl.ds(base, GRP)], dst_buf)

            def remap_row(j, _):
                for q in range(GCH // 16):
                    v = dst_buf[j, pl.ds(q * 16, 16)]
                    vl = v - lo
                    owned = (vl >= 0) & (vl < N_DEG)
                    dstl_buf[j, pl.ds(q * 16, 16)] = jnp.where(
                        owned, vl, jnp.full((16,), N_DEG, jnp.int32))
                return ()
            lax.fori_loop(0, GRP, remap_row, ())

            hd = [pltpu.async_copy(ones16, acc_deg.at[dstl_buf.at[j]],
                                   sem_d, add=True)
                  for j in range(GRP)]

            bufs = (rows_a, rows_b, rows_c)
            gsems = (sem_ga, sem_gb, sem_gc)
            psems = (sem_pa, sem_pb, sem_pc)
            hg = [None] * GRP
            hs = [None] * GRP
            hg[0] = pltpu.async_copy(h_ref.at[src_buf.at[0]], bufs[0],
                                     gsems[0])
            hg[1] = pltpu.async_copy(h_ref.at[src_buf.at[1]], bufs[1],
                                     gsems[1])
            for j in range(GRP):
                p = j % 3
                if j + 2 < GRP:
                    if j - 1 >= 0:
                        hs[j - 1].wait()
                    q = (j + 2) % 3
                    hg[j + 2] = pltpu.async_copy(
                        h_ref.at[src_buf.at[j + 2]], bufs[q], gsems[q])
                hg[j].wait()
                if do_square:
                    _square_rows(bufs[p], GCH)
                hs[j] = pltpu.async_copy(
                    bufs[p], acc_main.at[dst_buf.at[j]], psems[p], add=True)
            for j in range(max(GRP - 3, 0), GRP):
                hs[j].wait()
            for h_ in hd:
                h_.wait()
            return ()
        lax.fori_loop(0, NGRP, group_body, ())

        plsc.subcore_barrier()

        out_ref = sq_out if do_square else s_out
        pltpu.sync_copy(acc_main.at[pl.ds(r0, N_TILE)],
                        out_ref.at[pl.ds(r0, N_TILE)])
        pltpu.sync_copy(acc_deg.at[pl.ds(d0, DEG_TILE)],
                        deg_out.at[pl.ds(lo + d0, DEG_TILE)])

    @pl.when(c == 0)
    def _():
        run_core(False)

    @pl.when(c == 1)
    def _():
        run_core(True)


def _sc_aggregate(h, src2d, dst2d):
    mesh = plsc.VectorSubcoreMesh(core_axis_name="c", subcore_axis_name="s")
    kfn = pl.kernel(
        _sc_body,
        out_type=[
            jax.ShapeDtypeStruct((N_OUT, H), jnp.float32),
            jax.ShapeDtypeStruct((N_OUT, H), jnp.float32),
            jax.ShapeDtypeStruct((N_OUT, 16), jnp.float32),
        ],
        mesh=mesh,
        scratch_types=[
            pltpu.VMEM_SHARED((ACC_ROWS, H), jnp.float32),
            pltpu.VMEM_SHARED((DEG_ROWS, 16), jnp.float32),
            pltpu.VMEM((GRP, GCH), jnp.int32),
            pltpu.VMEM((GRP, GCH), jnp.int32),
            pltpu.VMEM((GRP, GCH), jnp.int32),
            pltpu.VMEM((GCH, H), jnp.float32),
            pltpu.VMEM((GCH, H), jnp.float32),
            pltpu.VMEM((GCH, H), jnp.float32),
            pltpu.VMEM((GCH, 16), jnp.float32),
            pltpu.SemaphoreType.DMA,
            pltpu.SemaphoreType.DMA,
            pltpu.SemaphoreType.DMA,
            pltpu.SemaphoreType.DMA,
            pltpu.SemaphoreType.DMA,
            pltpu.SemaphoreType.DMA,
            pltpu.SemaphoreType.DMA,
        ],
    )
    return kfn(h, src2d, dst2d)


def _post_kernel(s_ref, sq_ref, deg_ref, wm_ref, bm_ref, g_ref, b_ref, o_ref):
    s = s_ref[...]
    sq = sq_ref[...]
    deg = deg_ref[...][:, 0:1]
    deg_c = jnp.maximum(deg, 1.0)
    inv = 1.0 / deg_c
    mean = s * inv
    var = jnp.maximum(sq * inv - mean * mean, 0.0)
    std = jnp.sqrt(var + 1e-5)
    log_deg1 = jnp.log(deg + 1.0)
    scl_amp = log_deg1 * (1.0 / max(AVG_LOG_DEG, 1e-6))
    scl_att = AVG_LOG_DEG / jnp.maximum(log_deg1, 1e-6)
    scls = (None, scl_amp, scl_att)

    y = bm_ref[...][None, :]
    idx = 0
    for a in (mean, s, std):
        for sc in scls:
            m = a if sc is None else a * sc
            w = wm_ref[pl.ds(idx * H, H), :]
            y = y + jnp.dot(m, w, preferred_element_type=jnp.float32,
                            precision=jax.lax.Precision.HIGHEST)
            idx += 1

    mu = jnp.mean(y, axis=-1, keepdims=True)
    v = jnp.mean((y - mu) ** 2, axis=-1, keepdims=True)
    y = (y - mu) * jax.lax.rsqrt(v + 1e-5) * g_ref[...][None, :] + b_ref[...][None, :]
    o_ref[...] = jnp.maximum(y, 0.0)


def _post_mix(s, sq, deg, W_mix, b_mix, ln_g, ln_b):
    blk = 1000
    grid = (N // blk,)
    cat = W_mix.shape[0]
    return pl.pallas_call(
        _post_kernel,
        grid=grid,
        in_specs=[
            pl.BlockSpec((blk, H), lambda i: (i, 0)),
            pl.BlockSpec((blk, H), lambda i: (i, 0)),
            pl.BlockSpec((blk, 16), lambda i: (i, 0)),
            pl.BlockSpec((cat, OUT), lambda i: (0, 0)),
            pl.BlockSpec((OUT,), lambda i: (0,)),
            pl.BlockSpec((OUT,), lambda i: (0,)),
            pl.BlockSpec((OUT,), lambda i: (0,)),
        ],
        out_specs=pl.BlockSpec((blk, OUT), lambda i: (i, 0)),
        out_shape=jax.ShapeDtypeStruct((N, OUT), jnp.float32),
    )(s, sq, deg, W_mix, b_mix, ln_g, ln_b)


@jax.jit
def kernel(x, edge_index, W_pre, b_pre, W_mix, b_mix, ln_g, ln_b):
    src = edge_index[0]
    dst = edge_index[1]
    pad = E_PAD - E
    src_p = jnp.concatenate([src, jnp.zeros((pad,), jnp.int32)])
    dst_p = jnp.concatenate([dst, jnp.full((pad,), N_OUT, jnp.int32)])
    src2d = src_p.reshape(IDX_ROWS, GCH)
    dst2d = dst_p.reshape(IDX_ROWS, GCH)

    h = _pre_project(x, W_pre, b_pre)
    s, sq, deg = _sc_aggregate(h, src2d, dst2d)
    return _post_mix(s[:N], sq[:N], deg[:N], W_mix, b_mix, ln_g, ln_b)


# speedup vs baseline: 1.1847x; 1.1847x over previous
import math

import jax
import jax.numpy as jnp
from jax import lax
from jax.experimental import pallas as pl
from jax.experimental.pallas import tpu as pltpu
from jax.experimental.pallas import tpu_sc as plsc

N = 10000
E = 320000
D = 128
H = 128
OUT = 128
AVG_LOG_DEG = float((math.log(1.0) + math.log(2.0)) / 2.0)

NC, NS = 2, 16
GCH = 64
N_OUT = 10240
N_TILE = N_OUT // NS
ACC_ROWS = N_OUT + 8
N_DEG = N_OUT // NC
DEG_ROWS = N_DEG + NS
DEG_TILE = N_DEG // NS

CPT = 320
GRP = 16
NGRP = CPT // GRP
E_PAD = CPT * NS * GCH
IDX_ROWS = E_PAD // GCH


def _pre_kernel(x_ref, w_ref, b_ref, o_ref, o2_ref):
    acc = jnp.dot(x_ref[...], w_ref[...],
                  preferred_element_type=jnp.float32,
                  precision=jax.lax.Precision.HIGHEST)
    h = acc + b_ref[...][None, :]
    o_ref[...] = h
    o2_ref[...] = h * h


def _pre_project(x, W_pre, b_pre):
    blk = 1000
    grid = (N // blk,)
    return pl.pallas_call(
        _pre_kernel,
        grid=grid,
        in_specs=[
            pl.BlockSpec((blk, D), lambda i: (i, 0)),
            pl.BlockSpec((D, H), lambda i: (0, 0)),
            pl.BlockSpec((H,), lambda i: (0,)),
        ],
        out_specs=[pl.BlockSpec((blk, H), lambda i: (i, 0)),
                   pl.BlockSpec((blk, H), lambda i: (i, 0))],
        out_shape=[jax.ShapeDtypeStruct((N_OUT, H), jnp.float32),
                   jax.ShapeDtypeStruct((N_OUT, H), jnp.float32)],
    )(x, W_pre, b_pre)


def _sc_body(h_ref, h2_ref, src_ref, dst_ref, s_out, sq_out, deg_out,
             acc_main, acc_deg, src_buf, dst_buf, dstl_buf,
             rows_a, rows_b, ones16,
             sem_ga, sem_gb, sem_pa, sem_pb, sem_d):
    c = lax.axis_index("c")
    t = lax.axis_index("s")

    def fill_ones(i, _):
        ones16[i, :] = jnp.full((16,), 1.0, jnp.float32)
        return ()
    lax.fori_loop(0, GCH, fill_ones, (), unroll=4)

    r0 = t * N_TILE
    d0 = t * DEG_TILE
    lo = c * N_DEG
    dump = N_DEG + t

    def run_core(tab_ref, out_ref):
        def init_sub(k, _):
            rs = r0 + k * GCH
            pltpu.sync_copy(tab_ref.at[pl.ds(rs, GCH)], rows_a)
            pltpu.sync_copy(rows_a, acc_main.at[pl.ds(rs, GCH)])
            return ()
        lax.fori_loop(0, N_TILE // GCH, init_sub, ())

        def init_deg(k, _):
            pltpu.sync_copy(ones16.at[pl.ds(0, 64)],
                            acc_deg.at[pl.ds(d0 + k * 64, 64)])
            return ()
        lax.fori_loop(0, DEG_TILE // 64, init_deg, ())

        plsc.subcore_barrier()

        def group_body(grp, _):
            base = t * CPT + grp * GRP
            pltpu.sync_copy(src_ref.at[pl.ds(base, GRP)], src_buf)
            pltpu.sync_copy(dst_ref.at[pl.ds(base, GRP)], dst_buf)

            def remap_row(j, _):
                for q in range(GCH // 16):
                    v = dst_buf[j, pl.ds(q * 16, 16)]
                    vl = v - lo
                    owned = (vl >= 0) & (vl < N_DEG)
                    dstl_buf[j, pl.ds(q * 16, 16)] = jnp.where(
                        owned, vl, jnp.full((16,), 1, jnp.int32) * dump)
                return ()
            lax.fori_loop(0, GRP, remap_row, ())

            hd = [pltpu.async_copy(ones16, acc_deg.at[dstl_buf.at[j]],
                                   sem_d, add=True)
                  for j in range(GRP)]

            bufs = (rows_a, rows_b)
            gsems = (sem_ga, sem_gb)
            psems = (sem_pa, sem_pb)
            hg = [None] * GRP
            hs = [None] * GRP
            hg[0] = pltpu.async_copy(tab_ref.at[src_buf.at[0]], bufs[0],
                                     gsems[0])
            for j in range(GRP):
                p = j % 2
                if j + 1 < GRP:
                    if j - 1 >= 0:
                        hs[j - 1].wait()
                    q = (j + 1) % 2
                    hg[j + 1] = pltpu.async_copy(
                        tab_ref.at[src_buf.at[j + 1]], bufs[q], gsems[q])
                hg[j].wait()
                hs[j] = pltpu.async_copy(
                    bufs[p], acc_main.at[dst_buf.at[j]], psems[p], add=True)
            for j in range(max(GRP - 2, 0), GRP):
                hs[j].wait()
            for h_ in hd:
                h_.wait()
            return ()
        lax.fori_loop(0, NGRP, group_body, ())

        plsc.subcore_barrier()

        pltpu.sync_copy(acc_main.at[pl.ds(r0, N_TILE)],
                        out_ref.at[pl.ds(r0, N_TILE)])
        pltpu.sync_copy(acc_deg.at[pl.ds(d0, DEG_TILE)],
                        deg_out.at[pl.ds(lo + d0, DEG_TILE)])

    @pl.when(c == 0)
    def _():
        run_core(h_ref, s_out)

    @pl.when(c == 1)
    def _():
        run_core(h2_ref, sq_out)


def _sc_aggregate(h, h2, src2d, dst2d):
    mesh = plsc.VectorSubcoreMesh(core_axis_name="c", subcore_axis_name="s")
    kfn = pl.kernel(
        _sc_body,
        out_type=[
            jax.ShapeDtypeStruct((N_OUT, H), jnp.float32),
            jax.ShapeDtypeStruct((N_OUT, H), jnp.float32),
            jax.ShapeDtypeStruct((N_OUT, 16), jnp.float32),
        ],
        mesh=mesh,
        scratch_types=[
            pltpu.VMEM_SHARED((ACC_ROWS, H), jnp.float32),
            pltpu.VMEM_SHARED((DEG_ROWS, 16), jnp.float32),
            pltpu.VMEM((GRP, GCH), jnp.int32),
            pltpu.VMEM((GRP, GCH), jnp.int32),
            pltpu.VMEM((GRP, GCH), jnp.int32),
            pltpu.VMEM((GCH, H), jnp.float32),
            pltpu.VMEM((GCH, H), jnp.float32),
            pltpu.VMEM((GCH, 16), jnp.float32),
            pltpu.SemaphoreType.DMA,
            pltpu.SemaphoreType.DMA,
            pltpu.SemaphoreType.DMA,
            pltpu.SemaphoreType.DMA,
            pltpu.SemaphoreType.DMA,
        ],
    )
    return kfn(h, h2, src2d, dst2d)


def _post_kernel(s_ref, sq_ref, deg_ref, wm_ref, bm_ref, g_ref, b_ref, o_ref):
    s = s_ref[...]
    sq = sq_ref[...]
    deg = deg_ref[...][:, 0:1]
    deg_c = jnp.maximum(deg, 1.0)
    inv = 1.0 / deg_c
    mean = s * inv
    var = jnp.maximum(sq * inv - mean * mean, 0.0)
    std = jnp.sqrt(var + 1e-5)
    log_deg1 = jnp.log(deg + 1.0)
    scl_amp = log_deg1 * (1.0 / max(AVG_LOG_DEG, 1e-6))
    scl_att = AVG_LOG_DEG / jnp.maximum(log_deg1, 1e-6)
    scls = (None, scl_amp, scl_att)

    y = bm_ref[...][None, :]
    idx = 0
    for a in (mean, s, std):
        for sc in scls:
            m = a if sc is None else a * sc
            w = wm_ref[pl.ds(idx * H, H), :]
            y = y + jnp.dot(m, w, preferred_element_type=jnp.float32,
                            precision=jax.lax.Precision.HIGHEST)
            idx += 1

    mu = jnp.mean(y, axis=-1, keepdims=True)
    v = jnp.mean((y - mu) ** 2, axis=-1, keepdims=True)
    y = (y - mu) * jax.lax.rsqrt(v + 1e-5) * g_ref[...][None, :] + b_ref[...][None, :]
    o_ref[...] = jnp.maximum(y, 0.0)


def _post_mix(s, sq, deg, W_mix, b_mix, ln_g, ln_b):
    blk = 1000
    grid = (N // blk,)
    cat = W_mix.shape[0]
    return pl.pallas_call(
        _post_kernel,
        grid=grid,
        in_specs=[
            pl.BlockSpec((blk, H), lambda i: (i, 0)),
            pl.BlockSpec((blk, H), lambda i: (i, 0)),
            pl.BlockSpec((blk, 16), lambda i: (i, 0)),
            pl.BlockSpec((cat, OUT), lambda i: (0, 0)),
            pl.BlockSpec((OUT,), lambda i: (0,)),
            pl.BlockSpec((OUT,), lambda i: (0,)),
            pl.BlockSpec((OUT,), lambda i: (0,)),
        ],
        out_specs=pl.BlockSpec((blk, OUT), lambda i: (i, 0)),
        out_shape=jax.ShapeDtypeStruct((N, OUT), jnp.float32),
    )(s, sq, deg, W_mix, b_mix, ln_g, ln_b)


@jax.jit
def kernel(x, edge_index, W_pre, b_pre, W_mix, b_mix, ln_g, ln_b):
    src = edge_index[0]
    dst = edge_index[1]
    pad = E_PAD - E
    src_p = jnp.concatenate([src, jnp.zeros((pad,), jnp.int32)])
    dst_p = jnp.concatenate([dst, jnp.full((pad,), N_OUT, jnp.int32)])
    src2d = src_p.reshape(IDX_ROWS, GCH)
    dst2d = dst_p.reshape(IDX_ROWS, GCH)

    h, h2 = _pre_project(x, W_pre, b_pre)
    s, sq, deg = _sc_aggregate(h, h2, src2d, dst2d)
    return _post_mix(s[:N], sq[:N], deg[:N], W_mix, b_mix, ln_g, ln_b)


# speedup vs baseline: 1.2009x; 1.0137x over previous
import math

import jax
import jax.numpy as jnp
from jax import lax
from jax.experimental import pallas as pl
from jax.experimental.pallas import tpu as pltpu
from jax.experimental.pallas import tpu_sc as plsc

N = 10000
E = 320000
D = 128
H = 128
OUT = 128
AVG_LOG_DEG = float((math.log(1.0) + math.log(2.0)) / 2.0)

NC, NS = 2, 16
GCH = 64
N_OUT = 10240
N_TILE = N_OUT // NS
ACC_ROWS = N_OUT + 8
N_DEG = N_OUT // NC
DEG_ROWS = N_DEG + NS
DEG_TILE = N_DEG // NS

CPT = 320
GRP = 32
NGRP = CPT // GRP
E_PAD = CPT * NS * GCH
IDX_ROWS = E_PAD // GCH


def _pre_kernel(x_ref, w_ref, b_ref, o_ref, o2_ref):
    acc = jnp.dot(x_ref[...], w_ref[...],
                  preferred_element_type=jnp.float32,
                  precision=jax.lax.Precision.HIGHEST)
    h = acc + b_ref[...][None, :]
    o_ref[...] = h
    o2_ref[...] = h * h


def _pre_project(x, W_pre, b_pre):
    blk = 1000
    grid = (N // blk,)
    return pl.pallas_call(
        _pre_kernel,
        grid=grid,
        in_specs=[
            pl.BlockSpec((blk, D), lambda i: (i, 0)),
            pl.BlockSpec((D, H), lambda i: (0, 0)),
            pl.BlockSpec((H,), lambda i: (0,)),
        ],
        out_specs=[pl.BlockSpec((blk, H), lambda i: (i, 0)),
                   pl.BlockSpec((blk, H), lambda i: (i, 0))],
        out_shape=[jax.ShapeDtypeStruct((N_OUT, H), jnp.float32),
                   jax.ShapeDtypeStruct((N_OUT, H), jnp.float32)],
    )(x, W_pre, b_pre)


def _sc_body(h_ref, h2_ref, src_ref, dst_ref, s_out, sq_out, deg_out,
             acc_main, acc_deg, src_buf, dst_buf, dstl_buf,
             rows_a, rows_b, ones16,
             sem_ga, sem_gb, sem_pa, sem_pb, sem_d):
    c = lax.axis_index("c")
    t = lax.axis_index("s")

    def fill_ones(i, _):
        ones16[i, :] = jnp.full((16,), 1.0, jnp.float32)
        return ()
    lax.fori_loop(0, GCH, fill_ones, (), unroll=4)

    r0 = t * N_TILE
    d0 = t * DEG_TILE
    lo = c * N_DEG
    dump = N_DEG + t

    def run_core(tab_ref, out_ref):
        def init_sub(k, _):
            rs = r0 + k * GCH
            pltpu.sync_copy(tab_ref.at[pl.ds(rs, GCH)], rows_a)
            pltpu.sync_copy(rows_a, acc_main.at[pl.ds(rs, GCH)])
            return ()
        lax.fori_loop(0, N_TILE // GCH, init_sub, ())

        def init_deg(k, _):
            pltpu.sync_copy(ones16.at[pl.ds(0, 64)],
                            acc_deg.at[pl.ds(d0 + k * 64, 64)])
            return ()
        lax.fori_loop(0, DEG_TILE // 64, init_deg, ())

        plsc.subcore_barrier()

        def group_body(grp, _):
            base = t * CPT + grp * GRP
            pltpu.sync_copy(src_ref.at[pl.ds(base, GRP)], src_buf)
            pltpu.sync_copy(dst_ref.at[pl.ds(base, GRP)], dst_buf)

            def remap_row(j, _):
                for q in range(GCH // 16):
                    v = dst_buf[j, pl.ds(q * 16, 16)]
                    vl = v - lo
                    owned = (vl >= 0) & (vl < N_DEG)
                    dstl_buf[j, pl.ds(q * 16, 16)] = jnp.where(
                        owned, vl, jnp.full((16,), 1, jnp.int32) * dump)
                return ()
            lax.fori_loop(0, GRP, remap_row, ())

            hd = [pltpu.async_copy(ones16, acc_deg.at[dstl_buf.at[j]],
                                   sem_d, add=True)
                  for j in range(GRP)]

            bufs = (rows_a, rows_b)
            gsems = (sem_ga, sem_gb)
            psems = (sem_pa, sem_pb)
            hg = [None] * GRP
            hs = [None] * GRP
            hg[0] = pltpu.async_copy(tab_ref.at[src_buf.at[0]], bufs[0],
                                     gsems[0])
            for j in range(GRP):
                p = j % 2
                if j + 1 < GRP:
                    if j - 1 >= 0:
                        hs[j - 1].wait()
                    q = (j + 1) % 2
                    hg[j + 1] = pltpu.async_copy(
                        tab_ref.at[src_buf.at[j + 1]], bufs[q], gsems[q])
                hg[j].wait()
                hs[j] = pltpu.async_copy(
                    bufs[p], acc_main.at[dst_buf.at[j]], psems[p], add=True)
            for j in range(max(GRP - 2, 0), GRP):
                hs[j].wait()
            for h_ in hd:
                h_.wait()
            return ()
        lax.fori_loop(0, NGRP, group_body, ())

        plsc.subcore_barrier()

        pltpu.sync_copy(acc_main.at[pl.ds(r0, N_TILE)],
                        out_ref.at[pl.ds(r0, N_TILE)])
        pltpu.sync_copy(acc_deg.at[pl.ds(d0, DEG_TILE)],
                        deg_out.at[pl.ds(lo + d0, DEG_TILE)])

    @pl.when(c == 0)
    def _():
        run_core(h_ref, s_out)

    @pl.when(c == 1)
    def _():
        run_core(h2_ref, sq_out)


def _sc_aggregate(h, h2, src2d, dst2d):
    mesh = plsc.VectorSubcoreMesh(core_axis_name="c", subcore_axis_name="s")
    kfn = pl.kernel(
        _sc_body,
        out_type=[
            jax.ShapeDtypeStruct((N_OUT, H), jnp.float32),
            jax.ShapeDtypeStruct((N_OUT, H), jnp.float32),
            jax.ShapeDtypeStruct((N_OUT, 16), jnp.float32),
        ],
        mesh=mesh,
        scratch_types=[
            pltpu.VMEM_SHARED((ACC_ROWS, H), jnp.float32),
            pltpu.VMEM_SHARED((DEG_ROWS, 16), jnp.float32),
            pltpu.VMEM((GRP, GCH), jnp.int32),
            pltpu.VMEM((GRP, GCH), jnp.int32),
            pltpu.VMEM((GRP, GCH), jnp.int32),
            pltpu.VMEM((GCH, H), jnp.float32),
            pltpu.VMEM((GCH, H), jnp.float32),
            pltpu.VMEM((GCH, 16), jnp.float32),
            pltpu.SemaphoreType.DMA,
            pltpu.SemaphoreType.DMA,
            pltpu.SemaphoreType.DMA,
            pltpu.SemaphoreType.DMA,
            pltpu.SemaphoreType.DMA,
        ],
    )
    return kfn(h, h2, src2d, dst2d)


def _post_kernel(s_ref, sq_ref, deg_ref, wm_ref, bm_ref, g_ref, b_ref, o_ref):
    s = s_ref[...]
    sq = sq_ref[...]
    deg = deg_ref[...][:, 0:1]
    deg_c = jnp.maximum(deg, 1.0)
    inv = 1.0 / deg_c
    mean = s * inv
    var = jnp.maximum(sq * inv - mean * mean, 0.0)
    std = jnp.sqrt(var + 1e-5)
    log_deg1 = jnp.log(deg + 1.0)
    scl_amp = log_deg1 * (1.0 / max(AVG_LOG_DEG, 1e-6))
    scl_att = AVG_LOG_DEG / jnp.maximum(log_deg1, 1e-6)
    scls = (None, scl_amp, scl_att)

    y = bm_ref[...][None, :]
    idx = 0
    for a in (mean, s, std):
        for sc in scls:
            m = a if sc is None else a * sc
            w = wm_ref[pl.ds(idx * H, H), :]
            y = y + jnp.dot(m, w, preferred_element_type=jnp.float32,
                            precision=jax.lax.Precision.HIGHEST)
            idx += 1

    mu = jnp.mean(y, axis=-1, keepdims=True)
    v = jnp.mean((y - mu) ** 2, axis=-1, keepdims=True)
    y = (y - mu) * jax.lax.rsqrt(v + 1e-5) * g_ref[...][None, :] + b_ref[...][None, :]
    o_ref[...] = jnp.maximum(y, 0.0)


def _post_mix(s, sq, deg, W_mix, b_mix, ln_g, ln_b):
    blk = 1000
    grid = (N // blk,)
    cat = W_mix.shape[0]
    return pl.pallas_call(
        _post_kernel,
        grid=grid,
        in_specs=[
            pl.BlockSpec((blk, H), lambda i: (i, 0)),
            pl.BlockSpec((blk, H), lambda i: (i, 0)),
            pl.BlockSpec((blk, 16), lambda i: (i, 0)),
            pl.BlockSpec((cat, OUT), lambda i: (0, 0)),
            pl.BlockSpec((OUT,), lambda i: (0,)),
            pl.BlockSpec((OUT,), lambda i: (0,)),
            pl.BlockSpec((OUT,), lambda i: (0,)),
        ],
        out_specs=pl.BlockSpec((blk, OUT), lambda i: (i, 0)),
        out_shape=jax.ShapeDtypeStruct((N, OUT), jnp.float32),
    )(s, sq, deg, W_mix, b_mix, ln_g, ln_b)


@jax.jit
def kernel(x, edge_index, W_pre, b_pre, W_mix, b_mix, ln_g, ln_b):
    src = edge_index[0]
    dst = edge_index[1]
    pad = E_PAD - E
    src_p = jnp.concatenate([src, jnp.zeros((pad,), jnp.int32)])
    dst_p = jnp.concatenate([dst, jnp.full((pad,), N_OUT, jnp.int32)])
    src2d = src_p.reshape(IDX_ROWS, GCH)
    dst2d = dst_p.reshape(IDX_ROWS, GCH)

    h, h2 = _pre_project(x, W_pre, b_pre)
    s, sq, deg = _sc_aggregate(h, h2, src2d, dst2d)
    return _post_mix(s[:N], sq[:N], deg[:N], W_mix, b_mix, ln_g, ln_b)
